# baseline (speedup 1.0000x reference)
import functools

import jax
import jax.numpy as jnp
from jax import lax
from jax.experimental import pallas as pl
from jax.experimental.pallas import tpu as pltpu
from jax.experimental.pallas import tpu_sc as plsc

_LANES = 16
_BATCH = 4
_DEPTH = 16


def _make_gather(V, D, B):
    info = plsc.get_sparse_core_info()
    NC, NS = info.num_cores, info.num_subcores
    assert D == 2 * 8 and NC == 2
    b_per_w = B // NS
    assert b_per_w % _BATCH == 0 and B % (8 * NS) == 0
    nbatches = b_per_w // _BATCH
    mesh = plsc.VectorSubcoreMesh(core_axis_name="c", subcore_axis_name="s")

    @functools.partial(
        pl.kernel,
        mesh=mesh,
        out_type=jax.ShapeDtypeStruct((D, B), jnp.float32),
        scratch_types=[
            pltpu.VMEM((b_per_w + _LANES,), jnp.int32),
            pltpu.VMEM((_DEPTH, 8, _BATCH * 128), jnp.float32),
            pltpu.VMEM((8, b_per_w), jnp.float32),
        ]
        + [pltpu.SemaphoreType.DMA] * _DEPTH,
        compiler_params=pltpu.CompilerParams(needs_layout_passes=False),
    )
    def k(tableT_hbm, idx_hbm, outT_hbm, idx_v, ring_v, buf_v, *sems):
        jb = lax.axis_index("c")
        sid = lax.axis_index("s")
        jrow = jb * 8
        base = sid * b_per_w
        pltpu.sync_copy(idx_hbm.at[pl.ds(base, b_per_w)], idx_v.at[pl.ds(0, b_per_w)])
        lane = lax.iota(jnp.int32, _LANES)
        js = lane & 7
        hi = lane >= 8

        def issue(g, sem):
            par = g % _DEPTH
            for t0 in range(0, _BATCH, _LANES):
                rv = idx_v[pl.ds(g * _BATCH + t0, _LANES)]
                for t in range(min(_BATCH, _LANES)):
                    c = (rv[t] >> 7) * 128
                    pltpu.async_copy(
                        tableT_hbm.at[pl.ds(jrow, 8), pl.ds(c, 128)],
                        ring_v.at[par, :, pl.ds((t0 + t) * 128, 128)],
                        sem,
                    )

        def drain_extract(g, sem):
            par = g % _DEPTH
            pltpu.make_async_copy(
                tableT_hbm.at[pl.ds(0, 8), pl.ds(0, _BATCH * 128)],
                ring_v.at[par],
                sem,
            ).wait()
            for t0 in range(0, _BATCH, _LANES):
                rv = idx_v[pl.ds(g * _BATCH + t0, _LANES)]
                colv = rv & 127
                for t in range(0, min(_BATCH, _LANES), 2):
                    i = g * _BATCH + t0 + t
                    parv = jnp.full((_LANES,), par, jnp.int32)
                    cols = jnp.where(
                        hi,
                        jnp.full((_LANES,), (t0 + t + 1) * 128, jnp.int32)
                        + colv[t + 1],
                        jnp.full((_LANES,), (t0 + t) * 128, jnp.int32)
                        + colv[t],
                    )
                    ks = jnp.where(
                        hi,
                        jnp.full((_LANES,), i + 1, jnp.int32),
                        jnp.full((_LANES,), i, jnp.int32),
                    )
                    vals = plsc.load_gather(ring_v, [parv, js, cols])
                    plsc.store_scatter(buf_v, [js, ks], vals)

        for q in range(_DEPTH):
            issue(q, sems[q])

        def body(g, _):
            for q in range(_DEPTH):
                drain_extract(g + q, sems[q])

                @pl.when(g + q + _DEPTH < nbatches)
                def _():
                    issue(g + q + _DEPTH, sems[q])

            return 0

        lax.fori_loop(0, nbatches // _DEPTH, lambda p, c: body(p * _DEPTH, c), 0)
        pltpu.sync_copy(
            buf_v, outT_hbm.at[pl.ds(jrow, 8), pl.ds(base, b_per_w)]
        )

    return k


def kernel(batch, table):
    B = batch.shape[0]
    V, D = table.shape
    outT = _make_gather(V, D, B)(table.T, batch.astype(jnp.int32))
    return outT.T


# speedup vs baseline: 1.0724x; 1.0724x over previous
import functools

import jax
import jax.numpy as jnp
from jax import lax
from jax.experimental import pallas as pl
from jax.experimental.pallas import tpu as pltpu
from jax.experimental.pallas import tpu_sc as plsc

_LANES = 16
_BATCH = 8
_DEPTH = 8


def _make_gather(V, D, B):
    info = plsc.get_sparse_core_info()
    NC, NS = info.num_cores, info.num_subcores
    assert D == 2 * 8 and NC == 2
    b_per_w = B // NS
    assert b_per_w % _BATCH == 0 and B % (8 * NS) == 0
    nbatches = b_per_w // _BATCH
    mesh = plsc.VectorSubcoreMesh(core_axis_name="c", subcore_axis_name="s")

    @functools.partial(
        pl.kernel,
        mesh=mesh,
        out_type=jax.ShapeDtypeStruct((D, B), jnp.float32),
        scratch_types=[
            pltpu.VMEM((b_per_w + _LANES,), jnp.int32),
            pltpu.VMEM((_DEPTH, 8, _BATCH * 128), jnp.float32),
            pltpu.VMEM((8, b_per_w), jnp.float32),
        ]
        + [pltpu.SemaphoreType.DMA] * _DEPTH,
        compiler_params=pltpu.CompilerParams(needs_layout_passes=False),
    )
    def k(tableT_hbm, idx_hbm, outT_hbm, idx_v, ring_v, buf_v, *sems):
        jb = lax.axis_index("c")
        sid = lax.axis_index("s")
        jrow = jb * 8
        base = sid * b_per_w
        pltpu.sync_copy(idx_hbm.at[pl.ds(base, b_per_w)], idx_v.at[pl.ds(0, b_per_w)])
        lane = lax.iota(jnp.int32, _LANES)
        js = lane & 7
        hi = lane >= 8

        def issue(g, sem):
            par = g % _DEPTH
            for t0 in range(0, _BATCH, _LANES):
                rv = idx_v[pl.ds(g * _BATCH + t0, _LANES)]
                cv = (rv >> 7) * 128
                for t in range(min(_BATCH, _LANES)):
                    c = pl.multiple_of(cv[t], 128)
                    pltpu.async_copy(
                        tableT_hbm.at[pl.ds(jrow, 8), pl.ds(c, 128)],
                        ring_v.at[par, :, pl.ds((t0 + t) * 128, 128)],
                        sem,
                    )

        def drain_extract(g, sem):
            par = g % _DEPTH
            pltpu.make_async_copy(
                tableT_hbm.at[pl.ds(0, 8), pl.ds(0, _BATCH * 128)],
                ring_v.at[par],
                sem,
            ).wait()
            for t0 in range(0, _BATCH, _LANES):
                rv = idx_v[pl.ds(g * _BATCH + t0, _LANES)]
                colv = rv & 127
                for t in range(0, min(_BATCH, _LANES), 2):
                    i = g * _BATCH + t0 + t
                    parv = jnp.full((_LANES,), par, jnp.int32)
                    cols = jnp.where(
                        hi,
                        jnp.full((_LANES,), (t0 + t + 1) * 128, jnp.int32)
                        + colv[t + 1],
                        jnp.full((_LANES,), (t0 + t) * 128, jnp.int32)
                        + colv[t],
                    )
                    ks = jnp.where(
                        hi,
                        jnp.full((_LANES,), i + 1, jnp.int32),
                        jnp.full((_LANES,), i, jnp.int32),
                    )
                    vals = plsc.load_gather(ring_v, [parv, js, cols])
                    plsc.store_scatter(buf_v, [js, ks], vals)

        for q in range(_DEPTH):
            issue(q, sems[q])

        def body(g, _):
            for q in range(_DEPTH):
                drain_extract(g + q, sems[q])

                @pl.when(g + q + _DEPTH < nbatches)
                def _():
                    issue(g + q + _DEPTH, sems[q])

            return 0

        lax.fori_loop(0, nbatches // _DEPTH, lambda p, c: body(p * _DEPTH, c), 0)
        pltpu.sync_copy(
            buf_v, outT_hbm.at[pl.ds(jrow, 8), pl.ds(base, b_per_w)]
        )

    return k


def kernel(batch, table):
    B = batch.shape[0]
    V, D = table.shape
    outT = _make_gather(V, D, B)(table.T, batch.astype(jnp.int32))
    return outT.T
